# baseline (speedup 1.0000x reference)
import functools

import jax
import jax.numpy as jnp
from jax import lax
from jax.experimental import pallas as pl
from jax.experimental.pallas import tpu as pltpu
from jax.experimental.pallas import tpu_sc as plsc

_INTERPRET = False

N = 10000
E = 160000
D_IN = 128
C = 64
BASIS = 64
NORI = 12
L = 2
WF = 4
NG = 64
OUT = 1
RADIUS = 1000.0

R = E * NORI
R2 = N * NORI
BR = 1536
BN = 2000
BR2 = 2000
BP = 2000


def _gelu(t):
    return 0.5 * t * (1.0 + lax.erf(t * 0.7071067811865476))


def _call(body, **kw):
    return pl.pallas_call(body, interpret=_INTERPRET, **kw)


def _h0_body(x_ref, we_ref, o_ref):
    res = jnp.dot(x_ref[...], we_ref[...], preferred_element_type=jnp.float32)
    o_ref[...] = jnp.concatenate(
        [res, jnp.zeros((BN, 128 - C), jnp.float32)], axis=1)


def _h0(x, We):
    return _call(
        _h0_body,
        grid=(N // BN,),
        in_specs=[pl.BlockSpec((BN, D_IN), lambda i: (i, 0)),
                  pl.BlockSpec((D_IN, C), lambda i: (0, 0))],
        out_specs=pl.BlockSpec((BN, 128), lambda i: (i, 0)),
        out_shape=jax.ShapeDtypeStruct((N, 128), jnp.float32),
    )(x, We)


def _ori_body(oga_ref, ogb_ref, wo1_ref, bo1_ref, wo2_ref, bo2_ref,
              wfk0_ref, wfk1_ref, fk0_ref, fk1_ref):
    a = jnp.sum(oga_ref[...] * ogb_ref[...], axis=1, keepdims=True)
    pf = jnp.concatenate([a, a * a, a * a * a], axis=1)
    h1 = _gelu(jnp.dot(pf, wo1_ref[...],
                       preferred_element_type=jnp.float32) + bo1_ref[...])
    kb = _gelu(jnp.dot(h1, wo2_ref[...],
                       preferred_element_type=jnp.float32) + bo2_ref[...])
    fk0_ref[...] = jnp.dot(kb, wfk0_ref[...],
                           preferred_element_type=jnp.float32)
    fk1_ref[...] = jnp.dot(kb, wfk1_ref[...],
                           preferred_element_type=jnp.float32)


def _ori(ori_grid, Wo1, bo1, Wo2, bo2, Wfk):
    oga = jnp.repeat(ori_grid, NORI, axis=0)
    ogb = jnp.tile(ori_grid, (NORI, 1))
    full = lambda s: pl.BlockSpec(s, lambda: tuple(0 for _ in s))
    return _call(
        _ori_body,
        in_specs=[full((144, 3)), full((144, 3)),
                  full((3, C)), full((1, C)), full((C, BASIS)),
                  full((1, BASIS)), full((BASIS, C)), full((BASIS, C))],
        out_specs=[full((144, C)), full((144, C))],
        out_shape=[jax.ShapeDtypeStruct((144, C), jnp.float32)] * 2,
    )(oga, ogb, Wo1, bo1.reshape(1, C), Wo2, bo2.reshape(1, BASIS),
      Wfk[0], Wfk[1])


def _pf14(a, b):
    aa = a * a; ab = a * b; ba = b * a; bb = b * b
    return [a, b, aa, ab, ba, bb,
            aa * a, aa * b, ab * a, ab * b, ba * a, ba * b, bb * a, bb * b]


def _edge_body(rp_ref, oa_ref, ob_ref, wb1_ref, bb1_ref, wb2_ref, bb2_ref,
               wk_ref, m_ref):
    rp = rp_ref[...]
    d2 = jnp.sum(rp * rp, axis=1, keepdims=True)
    d = jnp.sqrt(d2)
    a1 = jnp.sum(rp * oa_ref[0], axis=1, keepdims=True)
    a2 = jnp.sum(rp * ob_ref[0], axis=1, keepdims=True)
    b1 = jnp.sqrt(jnp.clip(d2 - a1 * a1, 1e-8, None))
    b2 = jnp.sqrt(jnp.clip(d2 - a2 * a2, 1e-8, None))
    pf = jnp.concatenate(_pf14(a1, b1) + _pf14(a2, b2), axis=1)
    h1 = _gelu(jnp.dot(pf, wb1_ref[...],
                       preferred_element_type=jnp.float32) + bb1_ref[...])
    kb = _gelu(jnp.dot(h1, wb2_ref[...],
                       preferred_element_type=jnp.float32) + bb2_ref[...])
    u = d * (1.0 / RADIUS)
    u2 = u * u; u6 = u2 * u2 * u2
    env = 1.0 - 28.0 * u6 + 48.0 * u6 * u - 21.0 * u6 * u2
    win = jnp.where(d < RADIUS, env, 0.0)
    kbw = kb * win
    m_ref[...] = jnp.dot(kbw, wk_ref[...],
                         preferred_element_type=jnp.float32)


BE = 1280


def _edge(rel_pos, ori_grid, Wb1, bb1, Wb2, bb2, Wk_i):
    z = jnp.zeros((14, C), jnp.float32)
    wb1d = jnp.concatenate(
        [jnp.concatenate([Wb1, z], 1), jnp.concatenate([z, Wb1], 1)], 0)
    zc = jnp.zeros((C, C), jnp.float32)
    def bd(w):
        return jnp.concatenate(
            [jnp.concatenate([w, zc], 1), jnp.concatenate([zc, w], 1)], 0)
    nb = E // BE
    return _call(
        _edge_body,
        grid=(_NCH, nb),
        in_specs=[pl.BlockSpec((BE, 3), lambda q, i: (i, 0)),
                  pl.BlockSpec((1, 1, 3), lambda q, i: (2 * q, 0, 0)),
                  pl.BlockSpec((1, 1, 3), lambda q, i: (2 * q + 1, 0, 0)),
                  pl.BlockSpec((28, 2 * C), lambda q, i: (0, 0)),
                  pl.BlockSpec((1, 2 * C), lambda q, i: (0, 0)),
                  pl.BlockSpec((2 * C, 2 * BASIS), lambda q, i: (0, 0)),
                  pl.BlockSpec((1, 2 * BASIS), lambda q, i: (0, 0)),
                  pl.BlockSpec((2 * BASIS, 2 * C), lambda q, i: (0, 0))],
        out_specs=pl.BlockSpec((BE, 2 * C), lambda q, i: (q * nb + i, 0)),
        out_shape=jax.ShapeDtypeStruct((_NCH * E, 2 * C), jnp.float32),
    )(rel_pos, ori_grid.reshape(NORI, 1, 3), ori_grid.reshape(NORI, 1, 3),
      wb1d, jnp.tile(bb1, 2).reshape(1, 2 * C), bd(Wb2),
      jnp.tile(bb2, 2).reshape(1, 2 * BASIS), bd(Wk_i))


def _chunkify_body(h_ref, o_ref):
    o_ref[...] = h_ref[...]


def _chunkify(h_flat):
    return _call(
        _chunkify_body,
        grid=(_NCH, N // BN),
        in_specs=[pl.BlockSpec((BN, 128), lambda q, i: (i, q))],
        out_specs=pl.BlockSpec((BN, 128),
                               lambda q, i: (q * (N // BN) + i, 0)),
        out_shape=jax.ShapeDtypeStruct((_NCH * N, 128), jnp.float32),
    )(h_flat)


def _na_body(x10, x11, x12, x13, x14, x15, a_ref, cbt_ref, lgt_ref, lbt_ref,
             go_ref, gt_ref, x2n_ref):
    x1 = jnp.concatenate([r[...] for r in (x10, x11, x12, x13, x14, x15)],
                         axis=1)
    x2 = jnp.dot(x1, a_ref[...],
                 preferred_element_type=jnp.float32) + cbt_ref[...]
    mu = jnp.dot(jnp.dot(x2, go_ref[...], preferred_element_type=jnp.float32),
                 gt_ref[...], preferred_element_type=jnp.float32)
    dev = x2 - mu
    var = jnp.dot(jnp.dot(dev * dev, go_ref[...],
                          preferred_element_type=jnp.float32),
                  gt_ref[...], preferred_element_type=jnp.float32)
    x2n_ref[...] = dev * lax.rsqrt(var + 1e-5) * lgt_ref[...] + lbt_ref[...]


def _na(x1_chunks, A, cb_i, lg_i, lb_i, go, gt):
    F = NORI * C
    cbt = jnp.tile(cb_i, NORI).reshape(1, F)
    lgt = jnp.tile(lg_i, NORI).reshape(1, F)
    lbt = jnp.tile(lb_i, NORI).reshape(1, F)
    return _call(
        _na_body,
        grid=(N // BN,),
        in_specs=[pl.BlockSpec((BN, _FC), lambda i: (i, 0))] * _NCH +
                 [pl.BlockSpec((F, F), lambda i: (0, 0)),
                  pl.BlockSpec((1, F), lambda i: (0, 0)),
                  pl.BlockSpec((1, F), lambda i: (0, 0)),
                  pl.BlockSpec((1, F), lambda i: (0, 0)),
                  pl.BlockSpec((F, NORI), lambda i: (0, 0)),
                  pl.BlockSpec((NORI, F), lambda i: (0, 0))],
        out_specs=pl.BlockSpec((BN, F), lambda i: (i, 0)),
        out_shape=jax.ShapeDtypeStruct((N, F), jnp.float32),
    )(*x1_chunks, A, cbt, lgt, lbt, go, gt)


def _nb_res_body(x2_ref, hin_ref, w1_ref, b1_ref, w2_ref, b2_ref,
                 wr_ref, br_ref, h_ref, r_ref):
    t = _gelu(jnp.dot(x2_ref[...], w1_ref[...],
                      preferred_element_type=jnp.float32) + b1_ref[...])
    y = jnp.dot(t, w2_ref[...],
                preferred_element_type=jnp.float32) + b2_ref[...]
    h = y + hin_ref[...]
    h_ref[...] = h
    r_ref[...] = jnp.dot(h, wr_ref[...],
                         preferred_element_type=jnp.float32) + br_ref[...]


def _nb_nores_body(x2_ref, w1_ref, b1_ref, w2_ref, b2_ref,
                   wr_ref, br_ref, h_ref, r_ref):
    t = _gelu(jnp.dot(x2_ref[...], w1_ref[...],
                      preferred_element_type=jnp.float32) + b1_ref[...])
    h = jnp.dot(t, w2_ref[...],
                preferred_element_type=jnp.float32) + b2_ref[...]
    h_ref[...] = h
    r_ref[...] = jnp.dot(h, wr_ref[...],
                         preferred_element_type=jnp.float32) + br_ref[...]


def _nb(x2n_rows, hin_rows, W1_i, b1_i, W2_i, b2_i, Wr_i, br_i):
    H = WF * C
    w_specs = [pl.BlockSpec((C, H), lambda i: (0, 0)),
               pl.BlockSpec((1, H), lambda i: (0, 0)),
               pl.BlockSpec((H, C), lambda i: (0, 0)),
               pl.BlockSpec((1, C), lambda i: (0, 0)),
               pl.BlockSpec((C, OUT), lambda i: (0, 0)),
               pl.BlockSpec((1, OUT), lambda i: (0, 0))]
    row_spec = pl.BlockSpec((BR2, C), lambda i: (i, 0))
    args = [x2n_rows]
    in_specs = [row_spec]
    body = _nb_nores_body
    if hin_rows is not None:
        args.append(hin_rows)
        in_specs.append(row_spec)
        body = _nb_res_body
    args += [W1_i, b1_i.reshape(1, H), W2_i, b2_i.reshape(1, C),
             Wr_i, br_i.reshape(1, OUT)]
    return _call(
        body,
        grid=(R2 // BR2,),
        in_specs=in_specs + w_specs,
        out_specs=[row_spec, pl.BlockSpec((BR2, OUT), lambda i: (i, 0))],
        out_shape=[jax.ShapeDtypeStruct((R2, C), jnp.float32),
                   jax.ShapeDtypeStruct((R2, OUT), jnp.float32)],
    )(*args)


def _pool_body(r_ref, b_ref, o_ref):
    @pl.when(pl.program_id(0) == 0)
    def _():
        o_ref[...] = jnp.zeros_like(o_ref)
    scal = jnp.sum(r_ref[...], axis=1, keepdims=True) * (1.0 / (L * NORI))
    gids = lax.broadcasted_iota(jnp.int32, (NG, BP), 0)
    onehot = (b_ref[0] == gids).astype(jnp.float32)
    o_ref[...] += jnp.dot(onehot, scal, preferred_element_type=jnp.float32)


def _pool(rsum, batch):
    batch_r = batch.reshape(N // BP, 1, BP)
    return _call(
        _pool_body,
        grid=(N // BP,),
        in_specs=[pl.BlockSpec((BP, NORI), lambda i: (i, 0)),
                  pl.BlockSpec((1, 1, BP), lambda i: (i, 0, 0))],
        out_specs=pl.BlockSpec((NG, OUT), lambda i: (0, 0)),
        out_shape=jax.ShapeDtypeStruct((NG, OUT), jnp.float32),
    )(rsum, batch_r)


_NW = 32
_EW = E // _NW
_W0 = 200
_WL = 80
_ES = E // 16
_FC = 2 * C
_NCH = NORI * C // _FC
_NPAD = 10240
_NR = _NPAD // 16


def _sc_mesh():
    return plsc.VectorSubcoreMesh(core_axis_name="c", subcore_axis_name="s")


def _sc_pos_gather(pos, src, dst):
    posf = pos.reshape(N * 3)
    ngrp = _W0 // 16

    @functools.partial(
        pl.kernel,
        out_type=jax.ShapeDtypeStruct((E * 3,), jnp.float32),
        mesh=_sc_mesh(),
        compiler_params=pltpu.CompilerParams(needs_layout_passes=False),
        scratch_types=[
            pltpu.VMEM((N * 3,), jnp.float32),
            pltpu.VMEM((_W0 + 16,), jnp.int32),
            pltpu.VMEM((_W0 + 16,), jnp.int32),
            pltpu.VMEM((_W0 * 3,), jnp.float32),
        ],
    )
    def k(posf_hbm, src_hbm, dst_hbm, rel_hbm, posv, sidx, didx, relw):
        c = lax.axis_index("c")
        s = lax.axis_index("s")
        wid = s * 2 + c
        base = wid * _EW
        lane = lax.broadcasted_iota(jnp.int32, (16,), 0)
        tail_mask = lane < (_W0 - ngrp * 16)
        pltpu.sync_copy(posf_hbm, posv)

        def window(t, carry):
            off = base + t * _W0
            pltpu.sync_copy(src_hbm.at[pl.ds(off, _W0)], sidx.at[pl.ds(0, _W0)])
            pltpu.sync_copy(dst_hbm.at[pl.ds(off, _W0)], didx.at[pl.ds(0, _W0)])
            for j in range(ngrp + 1):
                msk = None if j < ngrp else tail_mask
                sv = sidx[pl.ds(j * 16, 16)]
                dv = didx[pl.ds(j * 16, 16)]
                if msk is not None:
                    sv = jnp.where(msk, sv, 0)
                    dv = jnp.where(msk, dv, 0)
                st_base = lane * 3 + j * 48
                for coord in range(3):
                    xs = plsc.load_gather(posv, [sv * 3 + coord], mask=msk)
                    xd = plsc.load_gather(posv, [dv * 3 + coord], mask=msk)
                    plsc.store_scatter(relw, [st_base + coord], xs - xd,
                                       mask=msk)
            pltpu.sync_copy(relw, rel_hbm.at[pl.ds(off * 3, _W0 * 3)])
            return carry

        lax.fori_loop(0, _EW // _W0, window, 0)

    return k(posf, src, dst)


def _sc_layer(h_tab, m_all, src, dst, broadcast_h):
    @functools.partial(
        pl.kernel,
        out_type=[jax.ShapeDtypeStruct((N, _FC), jnp.float32)] * _NCH,
        mesh=_sc_mesh(),
        scratch_types=[
            pltpu.VMEM((2, _WL), jnp.int32),
            pltpu.VMEM((2, _WL), jnp.int32),
            pltpu.VMEM((2, _WL, _FC), jnp.float32),
            pltpu.VMEM((2, _WL, _FC), jnp.float32),
            pltpu.VMEM((16, _FC), jnp.float32),
            pltpu.VMEM_SHARED((_NPAD, _FC), jnp.float32),
            pltpu.SemaphoreType.DMA,
            pltpu.SemaphoreType.DMA,
        ],
    )
    def k(h_hbm, src_hbm, dst_hbm, m_hbm,
          o0, o1, o2, o3, o4, o5,
          sidx, didx, mbuf, hbuf, zstrip, acc, semi, semg):
        c = lax.axis_index("c")
        s = lax.axis_index("s")
        zero16 = jnp.zeros((16,), jnp.float32)
        for i in range(16):
            for j in range(_FC // 16):
                zstrip[i, pl.ds(j * 16, 16)] = zero16
        outs = (o0, o1, o2, o3, o4, o5)
        row0 = pl.multiple_of(s * _NR, 128)

        def run_chunk(kc, out_hbm):
            def zrow(i, carry):
                pltpu.sync_copy(zstrip, acc.at[pl.ds(row0 + i * 16, 16), :])
                return carry
            lax.fori_loop(0, _NR // 16, zrow, 0)
            plsc.subcore_barrier()

            nwin = _ES // _WL

            def in_descs(t, slot):
                off = pl.multiple_of(s * _ES + t * _WL, 16)
                return (
                    pltpu.make_async_copy(src_hbm.at[pl.ds(off, _WL)],
                                          sidx.at[slot], semi),
                    pltpu.make_async_copy(dst_hbm.at[pl.ds(off, _WL)],
                                          didx.at[slot], semi),
                    pltpu.make_async_copy(
                        m_hbm.at[pl.ds(kc * E + off, _WL), :],
                        mbuf.at[slot], semi),
                )

            def gather_desc(slot):
                return pltpu.make_async_copy(h_hbm.at[sidx.at[slot]],
                                             hbuf.at[slot], semg)

            def prep_gather(slot):
                for d in in_descs(0, slot):
                    d.wait()
                if not broadcast_h:
                    for g in range(_WL // 16):
                        sidx[slot, pl.ds(g * 16, 16)] = (
                            sidx[slot, pl.ds(g * 16, 16)] + kc * N)
                gather_desc(slot).start()

            for d in in_descs(0, 0):
                d.start()
            prep_gather(0)
            for d in in_descs(1, 1):
                d.start()

            def window(t, carry):
                b = lax.rem(t, 2)
                nb = 1 - b
                gather_desc(b).wait()

                @pl.when(t + 1 < nwin)
                def _():
                    prep_gather(nb)
                def mulrow(w, carry2):
                    if broadcast_h:
                        hvs = [hbuf[b, w, pl.ds(g * 16, 16)] for g in range(4)]
                        for j in range(_FC // 16):
                            mv = mbuf[b, w, pl.ds(j * 16, 16)]
                            hbuf[b, w, pl.ds(j * 16, 16)] = hvs[j % 4] * mv
                    else:
                        for j in range(_FC // 16):
                            mv = mbuf[b, w, pl.ds(j * 16, 16)]
                            hbuf[b, w, pl.ds(j * 16, 16)] = (
                                hbuf[b, w, pl.ds(j * 16, 16)] * mv)
                    return carry2
                lax.fori_loop(0, _WL, mulrow, 0)
                pltpu.sync_copy(hbuf.at[b], acc.at[didx.at[b]], add=True)

                @pl.when(t + 2 < nwin)
                def _():
                    for d in in_descs(t + 2, b):
                        d.start()
                return carry

            lax.fori_loop(0, nwin, window, 0)
            plsc.subcore_barrier()

            @pl.when(s < 15)
            def _():
                pltpu.sync_copy(acc.at[pl.ds(row0, _NR), :],
                                out_hbm.at[pl.ds(row0, _NR), :])

            @pl.when(s == 15)
            def _():
                pltpu.sync_copy(acc.at[pl.ds(15 * _NR, N - 15 * _NR), :],
                                out_hbm.at[pl.ds(15 * _NR, N - 15 * _NR), :])

        for kc in range(_NCH):
            @pl.when(c == kc // 3)
            def _(kc=kc):
                run_chunk(kc, outs[kc])

    return k(h_tab, src, dst, m_all)


def kernel(x, pos, edge_index, batch, ori_grid, Wb1, bb1, Wb2, bb2,
           Wo1, bo1, Wo2, bo2, We, Wk, Wfk, cb, lg, lb, W1, b1, W2, b2,
           Wr, br):
    src, dst = edge_index[0], edge_index[1]

    rel_pos = _sc_pos_gather(pos, src, dst).reshape(E, 3)

    h0 = _h0(x, We)
    fk0, fk1 = _ori(ori_grid, Wo1, bo1, Wo2, bo2, Wfk)
    m0 = _edge(rel_pos, ori_grid, Wb1, bb1, Wb2, bb2, Wk[0])
    m1 = _edge(rel_pos, ori_grid, Wb1, bb1, Wb2, bb2, Wk[1])

    eye = jnp.eye(C, dtype=jnp.float32)
    F = NORI * C

    def mix_matrix(fk):
        fk3 = fk.reshape(NORI, NORI, C)
        a4 = jnp.einsum('opc,dc->odpc', fk3, eye) * (1.0 / NORI)
        return a4.reshape(F, F)

    A0 = mix_matrix(fk0)
    A1 = mix_matrix(fk1)
    group = jnp.arange(F, dtype=jnp.int32) // C
    gids = jnp.arange(NORI, dtype=jnp.int32)
    gt = (group[None, :] == gids[:, None]).astype(jnp.float32)
    go = gt.T * (1.0 / C)

    x1c = _sc_layer(h0, m0, src, dst, broadcast_h=True)
    x2n = _na(x1c, A0, cb[0], lg[0], lb[0], go, gt)
    h1_rows, r0 = _nb(x2n.reshape(R2, C), None, W1[0], b1[0], W2[0], b2[0],
                      Wr[0], br[0])

    h1_tab2 = _chunkify(h1_rows.reshape(N, F))
    x1c2 = _sc_layer(h1_tab2, m1, src, dst, broadcast_h=False)
    x2nb = _na(x1c2, A1, cb[1], lg[1], lb[1], go, gt)
    _h2_rows, r1 = _nb(x2nb.reshape(R2, C), h1_rows, W1[1], b1[1], W2[1],
                       b2[1], Wr[1], br[1])

    rsum = (r0 + r1).reshape(N, NORI)
    return _pool(rsum, batch)


# speedup vs baseline: 1.5151x; 1.5151x over previous
import functools

import jax
import jax.numpy as jnp
from jax import lax
from jax.experimental import pallas as pl
from jax.experimental.pallas import tpu as pltpu
from jax.experimental.pallas import tpu_sc as plsc

_INTERPRET = False

N = 10000
E = 160000
D_IN = 128
C = 64
BASIS = 64
NORI = 12
L = 2
WF = 4
NG = 64
OUT = 1
RADIUS = 1000.0

R = E * NORI
R2 = N * NORI
BR = 1536
BN = 2000
BR2 = 2000
BP = 2000


def _gelu(t):
    return 0.5 * t * (1.0 + lax.erf(t * 0.7071067811865476))


def _call(body, **kw):
    return pl.pallas_call(body, interpret=_INTERPRET, **kw)


def _h0_body(x_ref, we_ref, o_ref):
    res = jnp.dot(x_ref[...], we_ref[...], preferred_element_type=jnp.float32)
    o_ref[...] = jnp.concatenate(
        [res, jnp.zeros((BN, 128 - C), jnp.float32)], axis=1)


def _h0(x, We):
    return _call(
        _h0_body,
        grid=(N // BN,),
        in_specs=[pl.BlockSpec((BN, D_IN), lambda i: (i, 0)),
                  pl.BlockSpec((D_IN, C), lambda i: (0, 0))],
        out_specs=pl.BlockSpec((BN, 128), lambda i: (i, 0)),
        out_shape=jax.ShapeDtypeStruct((N, 128), jnp.float32),
    )(x, We)


def _ori_body(oga_ref, ogb_ref, wo1_ref, bo1_ref, wo2_ref, bo2_ref,
              wfk0_ref, wfk1_ref, fk0_ref, fk1_ref):
    a = jnp.sum(oga_ref[...] * ogb_ref[...], axis=1, keepdims=True)
    pf = jnp.concatenate([a, a * a, a * a * a], axis=1)
    h1 = _gelu(jnp.dot(pf, wo1_ref[...],
                       preferred_element_type=jnp.float32) + bo1_ref[...])
    kb = _gelu(jnp.dot(h1, wo2_ref[...],
                       preferred_element_type=jnp.float32) + bo2_ref[...])
    fk0_ref[...] = jnp.dot(kb, wfk0_ref[...],
                           preferred_element_type=jnp.float32)
    fk1_ref[...] = jnp.dot(kb, wfk1_ref[...],
                           preferred_element_type=jnp.float32)


def _ori(ori_grid, Wo1, bo1, Wo2, bo2, Wfk):
    oga = jnp.repeat(ori_grid, NORI, axis=0)
    ogb = jnp.tile(ori_grid, (NORI, 1))
    full = lambda s: pl.BlockSpec(s, lambda: tuple(0 for _ in s))
    return _call(
        _ori_body,
        in_specs=[full((144, 3)), full((144, 3)),
                  full((3, C)), full((1, C)), full((C, BASIS)),
                  full((1, BASIS)), full((BASIS, C)), full((BASIS, C))],
        out_specs=[full((144, C)), full((144, C))],
        out_shape=[jax.ShapeDtypeStruct((144, C), jnp.float32)] * 2,
    )(oga, ogb, Wo1, bo1.reshape(1, C), Wo2, bo2.reshape(1, BASIS),
      Wfk[0], Wfk[1])


def _pf14(a, b):
    aa = a * a; ab = a * b; ba = b * a; bb = b * b
    return [a, b, aa, ab, ba, bb,
            aa * a, aa * b, ab * a, ab * b, ba * a, ba * b, bb * a, bb * b]


def _edge_body(rp_ref, oa_ref, ob_ref, wb1_ref, bb1_ref, wb2_ref, bb2_ref,
               wk_ref, kb_ref, m_ref):
    rp = rp_ref[...]
    d2 = jnp.sum(rp * rp, axis=1, keepdims=True)
    d = jnp.sqrt(d2)
    a1 = jnp.sum(rp * oa_ref[0], axis=1, keepdims=True)
    a2 = jnp.sum(rp * ob_ref[0], axis=1, keepdims=True)
    b1 = jnp.sqrt(jnp.clip(d2 - a1 * a1, 1e-8, None))
    b2 = jnp.sqrt(jnp.clip(d2 - a2 * a2, 1e-8, None))
    pf = jnp.concatenate(_pf14(a1, b1) + _pf14(a2, b2), axis=1)
    h1 = _gelu(jnp.dot(pf, wb1_ref[...],
                       preferred_element_type=jnp.float32) + bb1_ref[...])
    kb = _gelu(jnp.dot(h1, wb2_ref[...],
                       preferred_element_type=jnp.float32) + bb2_ref[...])
    u = d * (1.0 / RADIUS)
    u2 = u * u; u6 = u2 * u2 * u2
    env = 1.0 - 28.0 * u6 + 48.0 * u6 * u - 21.0 * u6 * u2
    win = jnp.where(d < RADIUS, env, 0.0)
    kbw = kb * win
    kb_ref[...] = kbw
    m_ref[...] = jnp.dot(kbw, wk_ref[...],
                         preferred_element_type=jnp.float32)


BE = 1280


def _bd2(w):
    zc = jnp.zeros(w.shape, jnp.float32)
    return jnp.concatenate(
        [jnp.concatenate([w, zc], 1), jnp.concatenate([zc, w], 1)], 0)


def _edge(rel_pos, ori_grid, Wb1, bb1, Wb2, bb2, Wk_i):
    z = jnp.zeros((14, C), jnp.float32)
    wb1d = jnp.concatenate(
        [jnp.concatenate([Wb1, z], 1), jnp.concatenate([z, Wb1], 1)], 0)
    nb = E // BE
    return _call(
        _edge_body,
        grid=(_NCH, nb),
        in_specs=[pl.BlockSpec((BE, 3), lambda q, i: (i, 0)),
                  pl.BlockSpec((1, 1, 3), lambda q, i: (2 * q, 0, 0)),
                  pl.BlockSpec((1, 1, 3), lambda q, i: (2 * q + 1, 0, 0)),
                  pl.BlockSpec((28, 2 * C), lambda q, i: (0, 0)),
                  pl.BlockSpec((1, 2 * C), lambda q, i: (0, 0)),
                  pl.BlockSpec((2 * C, 2 * BASIS), lambda q, i: (0, 0)),
                  pl.BlockSpec((1, 2 * BASIS), lambda q, i: (0, 0)),
                  pl.BlockSpec((2 * BASIS, 2 * C), lambda q, i: (0, 0))],
        out_specs=[pl.BlockSpec((BE, 2 * C), lambda q, i: (q * nb + i, 0)),
                   pl.BlockSpec((BE, 2 * C), lambda q, i: (q * nb + i, 0))],
        out_shape=[jax.ShapeDtypeStruct((_NCH * E, 2 * C), jnp.float32)] * 2,
    )(rel_pos, ori_grid.reshape(NORI, 1, 3), ori_grid.reshape(NORI, 1, 3),
      wb1d, jnp.tile(bb1, 2).reshape(1, 2 * C), _bd2(Wb2),
      jnp.tile(bb2, 2).reshape(1, 2 * BASIS), _bd2(Wk_i))


def _mk_body(kb_ref, wk_ref, m_ref):
    m_ref[...] = jnp.dot(kb_ref[...], wk_ref[...],
                         preferred_element_type=jnp.float32)


def _m_from_kb(kbw, Wk_i):
    nb = _NCH * E // BE
    return _call(
        _mk_body,
        grid=(nb,),
        in_specs=[pl.BlockSpec((BE, 2 * C), lambda i: (i, 0)),
                  pl.BlockSpec((2 * BASIS, 2 * C), lambda i: (0, 0))],
        out_specs=pl.BlockSpec((BE, 2 * C), lambda i: (i, 0)),
        out_shape=jax.ShapeDtypeStruct((_NCH * E, 2 * C), jnp.float32),
    )(kbw, _bd2(Wk_i))


def _chunkify_body(h_ref, o_ref):
    o_ref[...] = h_ref[...]


def _chunkify(h_flat):
    return _call(
        _chunkify_body,
        grid=(_NCH, N // BN),
        in_specs=[pl.BlockSpec((BN, 128), lambda q, i: (i, q))],
        out_specs=pl.BlockSpec((BN, 128),
                               lambda q, i: (q * (N // BN) + i, 0)),
        out_shape=jax.ShapeDtypeStruct((_NCH * N, 128), jnp.float32),
    )(h_flat)


def _na_body(x10, x11, x12, x13, x14, x15, a_ref, cbt_ref, lgt_ref, lbt_ref,
             go_ref, gt_ref, x2n_ref):
    x1 = jnp.concatenate([r[...] for r in (x10, x11, x12, x13, x14, x15)],
                         axis=1)
    x2 = jnp.dot(x1, a_ref[...],
                 preferred_element_type=jnp.float32) + cbt_ref[...]
    mu = jnp.dot(jnp.dot(x2, go_ref[...], preferred_element_type=jnp.float32),
                 gt_ref[...], preferred_element_type=jnp.float32)
    dev = x2 - mu
    var = jnp.dot(jnp.dot(dev * dev, go_ref[...],
                          preferred_element_type=jnp.float32),
                  gt_ref[...], preferred_element_type=jnp.float32)
    x2n_ref[...] = dev * lax.rsqrt(var + 1e-5) * lgt_ref[...] + lbt_ref[...]


def _na(x1_chunks, A, cb_i, lg_i, lb_i, go, gt):
    F = NORI * C
    cbt = jnp.tile(cb_i, NORI).reshape(1, F)
    lgt = jnp.tile(lg_i, NORI).reshape(1, F)
    lbt = jnp.tile(lb_i, NORI).reshape(1, F)
    return _call(
        _na_body,
        grid=(N // BN,),
        in_specs=[pl.BlockSpec((BN, _FC), lambda i: (i, 0))] * _NCH +
                 [pl.BlockSpec((F, F), lambda i: (0, 0)),
                  pl.BlockSpec((1, F), lambda i: (0, 0)),
                  pl.BlockSpec((1, F), lambda i: (0, 0)),
                  pl.BlockSpec((1, F), lambda i: (0, 0)),
                  pl.BlockSpec((F, NORI), lambda i: (0, 0)),
                  pl.BlockSpec((NORI, F), lambda i: (0, 0))],
        out_specs=pl.BlockSpec((BN, F), lambda i: (i, 0)),
        out_shape=jax.ShapeDtypeStruct((N, F), jnp.float32),
    )(*x1_chunks, A, cbt, lgt, lbt, go, gt)


def _nb_res_body(x2_ref, hin_ref, w1_ref, b1_ref, w2_ref, b2_ref,
                 wr_ref, br_ref, h_ref, r_ref):
    t = _gelu(jnp.dot(x2_ref[...], w1_ref[...],
                      preferred_element_type=jnp.float32) + b1_ref[...])
    y = jnp.dot(t, w2_ref[...],
                preferred_element_type=jnp.float32) + b2_ref[...]
    h = y + hin_ref[...]
    h_ref[...] = h
    r_ref[...] = jnp.dot(h, wr_ref[...],
                         preferred_element_type=jnp.float32) + br_ref[...]


def _nb_nores_body(x2_ref, w1_ref, b1_ref, w2_ref, b2_ref,
                   wr_ref, br_ref, h_ref, r_ref):
    t = _gelu(jnp.dot(x2_ref[...], w1_ref[...],
                      preferred_element_type=jnp.float32) + b1_ref[...])
    h = jnp.dot(t, w2_ref[...],
                preferred_element_type=jnp.float32) + b2_ref[...]
    h_ref[...] = h
    r_ref[...] = jnp.dot(h, wr_ref[...],
                         preferred_element_type=jnp.float32) + br_ref[...]


def _nb(x2n_rows, hin_rows, W1_i, b1_i, W2_i, b2_i, Wr_i, br_i):
    H = WF * C
    w_specs = [pl.BlockSpec((C, H), lambda i: (0, 0)),
               pl.BlockSpec((1, H), lambda i: (0, 0)),
               pl.BlockSpec((H, C), lambda i: (0, 0)),
               pl.BlockSpec((1, C), lambda i: (0, 0)),
               pl.BlockSpec((C, OUT), lambda i: (0, 0)),
               pl.BlockSpec((1, OUT), lambda i: (0, 0))]
    row_spec = pl.BlockSpec((BR2, C), lambda i: (i, 0))
    args = [x2n_rows]
    in_specs = [row_spec]
    body = _nb_nores_body
    if hin_rows is not None:
        args.append(hin_rows)
        in_specs.append(row_spec)
        body = _nb_res_body
    args += [W1_i, b1_i.reshape(1, H), W2_i, b2_i.reshape(1, C),
             Wr_i, br_i.reshape(1, OUT)]
    return _call(
        body,
        grid=(R2 // BR2,),
        in_specs=in_specs + w_specs,
        out_specs=[row_spec, pl.BlockSpec((BR2, OUT), lambda i: (i, 0))],
        out_shape=[jax.ShapeDtypeStruct((R2, C), jnp.float32),
                   jax.ShapeDtypeStruct((R2, OUT), jnp.float32)],
    )(*args)


def _pool_body(r_ref, b_ref, o_ref):
    @pl.when(pl.program_id(0) == 0)
    def _():
        o_ref[...] = jnp.zeros_like(o_ref)
    scal = jnp.sum(r_ref[...], axis=1, keepdims=True) * (1.0 / (L * NORI))
    gids = lax.broadcasted_iota(jnp.int32, (NG, BP), 0)
    onehot = (b_ref[0] == gids).astype(jnp.float32)
    o_ref[...] += jnp.dot(onehot, scal, preferred_element_type=jnp.float32)


def _pool(rsum, batch):
    batch_r = batch.reshape(N // BP, 1, BP)
    return _call(
        _pool_body,
        grid=(N // BP,),
        in_specs=[pl.BlockSpec((BP, NORI), lambda i: (i, 0)),
                  pl.BlockSpec((1, 1, BP), lambda i: (i, 0, 0))],
        out_specs=pl.BlockSpec((NG, OUT), lambda i: (0, 0)),
        out_shape=jax.ShapeDtypeStruct((NG, OUT), jnp.float32),
    )(rsum, batch_r)


_NW = 32
_EW = E // _NW
_W0 = 200
_WL = 80
_ES = E // 16
_FC = 2 * C
_NCH = NORI * C // _FC
_NPAD = 10240
_NR = _NPAD // 16


def _sc_mesh():
    return plsc.VectorSubcoreMesh(core_axis_name="c", subcore_axis_name="s")


def _sc_pos_gather(pos, src, dst):
    posf = pos.reshape(N * 3)
    ngrp = _W0 // 16

    @functools.partial(
        pl.kernel,
        out_type=jax.ShapeDtypeStruct((E * 3,), jnp.float32),
        mesh=_sc_mesh(),
        compiler_params=pltpu.CompilerParams(needs_layout_passes=False),
        scratch_types=[
            pltpu.VMEM((N * 3,), jnp.float32),
            pltpu.VMEM((_W0 + 16,), jnp.int32),
            pltpu.VMEM((_W0 + 16,), jnp.int32),
            pltpu.VMEM((_W0 * 3,), jnp.float32),
        ],
    )
    def k(posf_hbm, src_hbm, dst_hbm, rel_hbm, posv, sidx, didx, relw):
        c = lax.axis_index("c")
        s = lax.axis_index("s")
        wid = s * 2 + c
        base = wid * _EW
        lane = lax.broadcasted_iota(jnp.int32, (16,), 0)
        tail_mask = lane < (_W0 - ngrp * 16)
        pltpu.sync_copy(posf_hbm, posv)

        def window(t, carry):
            off = base + t * _W0
            pltpu.sync_copy(src_hbm.at[pl.ds(off, _W0)], sidx.at[pl.ds(0, _W0)])
            pltpu.sync_copy(dst_hbm.at[pl.ds(off, _W0)], didx.at[pl.ds(0, _W0)])
            for j in range(ngrp + 1):
                msk = None if j < ngrp else tail_mask
                sv = sidx[pl.ds(j * 16, 16)]
                dv = didx[pl.ds(j * 16, 16)]
                if msk is not None:
                    sv = jnp.where(msk, sv, 0)
                    dv = jnp.where(msk, dv, 0)
                st_base = lane * 3 + j * 48
                for coord in range(3):
                    xs = plsc.load_gather(posv, [sv * 3 + coord], mask=msk)
                    xd = plsc.load_gather(posv, [dv * 3 + coord], mask=msk)
                    plsc.store_scatter(relw, [st_base + coord], xs - xd,
                                       mask=msk)
            pltpu.sync_copy(relw, rel_hbm.at[pl.ds(off * 3, _W0 * 3)])
            return carry

        lax.fori_loop(0, _EW // _W0, window, 0)

    return k(posf, src, dst)


def _sc_layer(h_tab, m_all, src, dst, broadcast_h):
    @functools.partial(
        pl.kernel,
        out_type=[jax.ShapeDtypeStruct((N, _FC), jnp.float32)] * _NCH,
        mesh=_sc_mesh(),
        scratch_types=[
            pltpu.VMEM((2, _WL), jnp.int32),
            pltpu.VMEM((2, _WL), jnp.int32),
            pltpu.VMEM((2, _WL, _FC), jnp.float32),
            pltpu.VMEM((2, _WL, _FC), jnp.float32),
            pltpu.VMEM((16, _FC), jnp.float32),
            pltpu.VMEM_SHARED((_NPAD, _FC), jnp.float32),
            pltpu.SemaphoreType.DMA,
            pltpu.SemaphoreType.DMA,
        ],
    )
    def k(h_hbm, src_hbm, dst_hbm, m_hbm,
          o0, o1, o2, o3, o4, o5,
          sidx, didx, mbuf, hbuf, zstrip, acc, semi, semg):
        c = lax.axis_index("c")
        s = lax.axis_index("s")
        zero16 = jnp.zeros((16,), jnp.float32)
        for i in range(16):
            for j in range(_FC // 16):
                zstrip[i, pl.ds(j * 16, 16)] = zero16
        outs = (o0, o1, o2, o3, o4, o5)
        row0 = pl.multiple_of(s * _NR, 128)

        def run_chunk(kc, out_hbm):
            def zrow(i, carry):
                pltpu.sync_copy(zstrip, acc.at[pl.ds(row0 + i * 16, 16), :])
                return carry
            lax.fori_loop(0, _NR // 16, zrow, 0)
            plsc.subcore_barrier()

            nwin = _ES // _WL

            def in_descs(t, slot):
                off = pl.multiple_of(s * _ES + t * _WL, 16)
                return (
                    pltpu.make_async_copy(src_hbm.at[pl.ds(off, _WL)],
                                          sidx.at[slot], semi),
                    pltpu.make_async_copy(dst_hbm.at[pl.ds(off, _WL)],
                                          didx.at[slot], semi),
                    pltpu.make_async_copy(
                        m_hbm.at[pl.ds(kc * E + off, _WL), :],
                        mbuf.at[slot], semi),
                )

            def gather_desc(slot):
                return pltpu.make_async_copy(h_hbm.at[sidx.at[slot]],
                                             hbuf.at[slot], semg)

            def prep_gather(slot):
                for d in in_descs(0, slot):
                    d.wait()
                if not broadcast_h:
                    for g in range(_WL // 16):
                        sidx[slot, pl.ds(g * 16, 16)] = (
                            sidx[slot, pl.ds(g * 16, 16)] + kc * N)
                gather_desc(slot).start()

            for d in in_descs(0, 0):
                d.start()
            prep_gather(0)
            for d in in_descs(1, 1):
                d.start()

            def do_compute(b):
                def mulrow(w, carry2):
                    if broadcast_h:
                        hvs = [hbuf[b, w, pl.ds(g * 16, 16)]
                               for g in range(4)]
                        for j in range(_FC // 16):
                            mv = mbuf[b, w, pl.ds(j * 16, 16)]
                            hbuf[b, w, pl.ds(j * 16, 16)] = hvs[j % 4] * mv
                    else:
                        for j in range(_FC // 16):
                            mv = mbuf[b, w, pl.ds(j * 16, 16)]
                            hbuf[b, w, pl.ds(j * 16, 16)] = (
                                hbuf[b, w, pl.ds(j * 16, 16)] * mv)
                    return carry2
                lax.fori_loop(0, _WL, mulrow, 0)
                pltpu.sync_copy(hbuf.at[b], acc.at[didx.at[b]], add=True)

            def window_body(t, b):
                gather_desc(b).wait()
                prep_gather(1 - b)
                do_compute(b)

                @pl.when(t + 2 < nwin)
                def _():
                    for dsc in in_descs(t + 2, b):
                        dsc.start()

            def pair(g, carry):
                t = g * 2
                window_body(t, 0)
                window_body(t + 1, 1)
                return carry

            lax.fori_loop(0, nwin // 2, pair, 0)
            gather_desc(0).wait()
            do_compute(0)
            plsc.subcore_barrier()

            @pl.when(s < 15)
            def _():
                pltpu.sync_copy(acc.at[pl.ds(row0, _NR), :],
                                out_hbm.at[pl.ds(row0, _NR), :])

            @pl.when(s == 15)
            def _():
                pltpu.sync_copy(acc.at[pl.ds(15 * _NR, N - 15 * _NR), :],
                                out_hbm.at[pl.ds(15 * _NR, N - 15 * _NR), :])

        for kc in range(_NCH):
            @pl.when(c == kc // 3)
            def _(kc=kc):
                run_chunk(kc, outs[kc])

    return k(h_tab, src, dst, m_all)


def kernel(x, pos, edge_index, batch, ori_grid, Wb1, bb1, Wb2, bb2,
           Wo1, bo1, Wo2, bo2, We, Wk, Wfk, cb, lg, lb, W1, b1, W2, b2,
           Wr, br):
    src, dst = edge_index[0], edge_index[1]

    rel_pos = _sc_pos_gather(pos, src, dst).reshape(E, 3)

    h0 = _h0(x, We)
    fk0, fk1 = _ori(ori_grid, Wo1, bo1, Wo2, bo2, Wfk)
    kbw, m0 = _edge(rel_pos, ori_grid, Wb1, bb1, Wb2, bb2, Wk[0])
    m1 = _m_from_kb(kbw, Wk[1])

    eye = jnp.eye(C, dtype=jnp.float32)
    F = NORI * C

    def mix_matrix(fk):
        fk3 = fk.reshape(NORI, NORI, C)
        a4 = jnp.einsum('opc,dc->odpc', fk3, eye) * (1.0 / NORI)
        return a4.reshape(F, F)

    A0 = mix_matrix(fk0)
    A1 = mix_matrix(fk1)
    group = jnp.arange(F, dtype=jnp.int32) // C
    gids = jnp.arange(NORI, dtype=jnp.int32)
    gt = (group[None, :] == gids[:, None]).astype(jnp.float32)
    go = gt.T * (1.0 / C)

    x1c = _sc_layer(h0, m0, src, dst, broadcast_h=True)
    x2n = _na(x1c, A0, cb[0], lg[0], lb[0], go, gt)
    h1_rows, r0 = _nb(x2n.reshape(R2, C), None, W1[0], b1[0], W2[0], b2[0],
                      Wr[0], br[0])

    h1_tab2 = _chunkify(h1_rows.reshape(N, F))
    x1c2 = _sc_layer(h1_tab2, m1, src, dst, broadcast_h=False)
    x2nb = _na(x1c2, A1, cb[1], lg[1], lb[1], go, gt)
    _h2_rows, r1 = _nb(x2nb.reshape(R2, C), h1_rows, W1[1], b1[1], W2[1],
                       b2[1], Wr[1], br[1])

    rsum = (r0 + r1).reshape(N, NORI)
    return _pool(rsum, batch)
